# baseline (speedup 1.0000x reference)
import functools

import jax
import jax.numpy as jnp
from jax import lax
from jax.experimental import pallas as pl
from jax.experimental.pallas import tpu as pltpu
from jax.experimental.pallas import tpu_sc as plsc

NC = 2
NS = 16
NW = NC * NS
L = 16

CHUNK = 128


def _pad_to(n, m):
    return ((n + m - 1) // m) * m


def _sc_mesh():
    return plsc.VectorSubcoreMesh(
        core_axis_name="c", subcore_axis_name="s", num_cores=NC, num_subcores=NS
    )


_SC_PARAMS = pltpu.CompilerParams(
    needs_layout_passes=False, use_tc_tiling_on_sc=False
)


def _make_deg_kernel(npad, ept):
    npass = 4
    seg = npad // npass
    cb = seg // NS
    dseg = NB * CHUNK
    dblk = ept // dseg

    @functools.partial(
        pl.kernel,
        out_type=jax.ShapeDtypeStruct((NC, npad), jnp.float32),
        mesh=_sc_mesh(),
        compiler_params=_SC_PARAMS,
        scratch_types=[
            pltpu.VMEM((npad,), jnp.float32),
            pltpu.VMEM((dseg,), jnp.int32),
            pltpu.VMEM_SHARED((NS, seg), jnp.float32),
            pltpu.VMEM((NS, cb), jnp.float32),
            pltpu.VMEM((cb,), jnp.float32),
        ],
    )
    def deg_kernel(dst_hbm, out_hbm, hist, didx, staging, colblk, summed):
        c = lax.axis_index("c")
        s = lax.axis_index("s")
        wid = c * NS + s

        zero16 = jnp.zeros((L,), jnp.float32)

        def zbody(i, _):
            hist[pl.ds(i * L, L)] = zero16
            return 0

        lax.fori_loop(0, npad // L, zbody, 0)

        ones16 = jnp.ones((L,), jnp.float32)

        def abody(i, _):
            d = didx[pl.ds(i * L, L)]
            plsc.addupdate_scatter(hist, [d], ones16)
            return 0

        def dbody(bb, _):
            pltpu.sync_copy(dst_hbm.at[wid * dblk + bb], didx)
            lax.fori_loop(0, dseg // L, abody, 0)
            return 0

        lax.fori_loop(0, dblk, dbody, 0)

        def rbody(i, _):
            v = colblk[0, pl.ds(i * L, L)]
            for t in range(1, NS):
                v = v + colblk[t, pl.ds(i * L, L)]
            summed[pl.ds(i * L, L)] = v
            return 0

        for p in range(npass):
            pltpu.sync_copy(hist.at[pl.ds(p * seg, seg)], staging.at[s])
            plsc.subcore_barrier()
            pltpu.sync_copy(staging.at[:, pl.ds(s * cb, cb)], colblk)
            lax.fori_loop(0, cb // L, rbody, 0)
            pltpu.sync_copy(
                summed, out_hbm.at[c, pl.ds(p * seg + s * cb, cb)])
            plsc.subcore_barrier()

    return deg_kernel


NB = 16


def _agg_block_pipeline(g_hbm, acc, sidx, didx, rows_a, rows_b,
                        gsem_a, gsem_b, ssem_a, ssem_b):
    def gath(j, buf, sem):
        pltpu.async_copy(g_hbm.at[sidx.at[j]], buf, sem)

    def scat(j, buf, sem):
        pltpu.async_copy(buf, acc.at[didx.at[j]], sem, add=True)

    def wait_g(buf, sem):
        pltpu.make_async_copy(g_hbm.at[sidx.at[0]], buf, sem).wait()

    def wait_s(buf, sem):
        pltpu.make_async_copy(buf, acc.at[didx.at[0]], sem).wait()

    gath(0, rows_a, gsem_a)
    gath(1, rows_b, gsem_b)
    wait_g(rows_a, gsem_a)
    scat(0, rows_a, ssem_a)
    wait_s(rows_a, ssem_a)
    gath(2, rows_a, gsem_a)
    wait_g(rows_b, gsem_b)
    scat(1, rows_b, ssem_b)

    def pbody(jj, _):
        j0 = 2 * jj
        wait_s(rows_b, ssem_b)
        gath(j0 + 1, rows_b, gsem_b)
        wait_g(rows_a, gsem_a)
        scat(j0, rows_a, ssem_a)
        wait_s(rows_a, ssem_a)
        gath(j0 + 2, rows_a, gsem_a)
        wait_g(rows_b, gsem_b)
        scat(j0 + 1, rows_b, ssem_b)
        return 0

    lax.fori_loop(1, NB // 2 - 1, pbody, 0)

    wait_s(rows_b, ssem_b)
    gath(NB - 1, rows_b, gsem_b)
    wait_g(rows_a, gsem_a)
    scat(NB - 2, rows_a, ssem_a)
    wait_g(rows_b, gsem_b)
    scat(NB - 1, rows_b, ssem_b)
    wait_s(rows_a, ssem_a)
    wait_s(rows_b, ssem_b)


def _zero_acc(acc, ztile, s, zrows, feat):
    zero16 = jnp.zeros((L,), jnp.float32)
    for i in range(L):
        for j in range(feat // L):
            ztile[i, pl.ds(j * L, L)] = zero16

    def zbody(i, _):
        pltpu.sync_copy(ztile, acc.at[pl.ds((s * zrows + i) * L, L), :])
        return 0

    lax.fori_loop(0, zrows, zbody, 0)


def _make_agg_kernel(npad, feat, cpt):
    zrows = npad // NS // L
    wb = 128
    wchunks = npad // NS // wb
    nblk = cpt // NB

    @functools.partial(
        pl.kernel,
        out_type=jax.ShapeDtypeStruct((NC, npad, feat), jnp.float32),
        mesh=_sc_mesh(),
        compiler_params=_SC_PARAMS,
        scratch_types=[
            pltpu.VMEM((NB, CHUNK), jnp.int32),
            pltpu.VMEM((NB, CHUNK), jnp.int32),
            pltpu.VMEM((CHUNK, feat), jnp.float32),
            pltpu.VMEM((CHUNK, feat), jnp.float32),
            pltpu.VMEM_SHARED((npad, feat), jnp.float32),
            pltpu.VMEM((L, feat), jnp.float32),
            pltpu.SemaphoreType.DMA,
            pltpu.SemaphoreType.DMA,
            pltpu.SemaphoreType.DMA,
            pltpu.SemaphoreType.DMA,
        ],
    )
    def agg_kernel(g_hbm, src_hbm, dst_hbm, out_hbm,
                   sidx, didx, rows_a, rows_b, acc, ztile,
                   gsem_a, gsem_b, ssem_a, ssem_b):
        c = lax.axis_index("c")
        s = lax.axis_index("s")
        wid = c * NS + s

        _zero_acc(acc, ztile, s, zrows, feat)
        plsc.subcore_barrier()

        def bbody(bb, _):
            pltpu.sync_copy(src_hbm.at[wid * nblk + bb], sidx)
            pltpu.sync_copy(dst_hbm.at[wid * nblk + bb], didx)
            def gbody(j, _):
                pltpu.async_copy(g_hbm.at[sidx.at[2 * j]], rows_a, gsem_a)
                pltpu.async_copy(g_hbm.at[sidx.at[2 * j + 1]], rows_b, gsem_b)
                pltpu.make_async_copy(g_hbm.at[sidx.at[0]], rows_a, gsem_a).wait()
                pltpu.make_async_copy(g_hbm.at[sidx.at[0]], rows_b, gsem_b).wait()
                return 0
            lax.fori_loop(0, NB // 2, gbody, 0)
            return 0

        lax.fori_loop(0, nblk, bbody, 0)
        plsc.subcore_barrier()

        def wbody(k, _):
            r0 = (s * wchunks + k) * wb
            pltpu.sync_copy(acc.at[pl.ds(r0, wb), :], rows_a)
            pltpu.sync_copy(rows_a, out_hbm.at[c, pl.ds(r0, wb), :])
            return 0

        lax.fori_loop(0, wchunks, wbody, 0)

    return agg_kernel


def _make_agg_split_kernel(npad, feat, cpt):
    half = npad // NC
    nacc = half + 256
    zrows = nacc // NS // L
    wrows = half // NS
    nblk = cpt // NB

    @functools.partial(
        pl.kernel,
        out_type=jax.ShapeDtypeStruct((NC, half, feat), jnp.float32),
        mesh=_sc_mesh(),
        compiler_params=_SC_PARAMS,
        scratch_types=[
            pltpu.VMEM((NB, CHUNK), jnp.int32),
            pltpu.VMEM((NB, CHUNK), jnp.int32),
            pltpu.VMEM((CHUNK, feat), jnp.float32),
            pltpu.VMEM((CHUNK, feat), jnp.float32),
            pltpu.VMEM_SHARED((nacc, feat), jnp.float32),
            pltpu.VMEM((L, feat), jnp.float32),
            pltpu.SemaphoreType.DMA,
            pltpu.SemaphoreType.DMA,
            pltpu.SemaphoreType.DMA,
            pltpu.SemaphoreType.DMA,
        ],
    )
    def agg_kernel(g_hbm, src_hbm, dst_hbm, out_hbm,
                   sidx, didx, rows_a, rows_b, acc, ztile,
                   gsem_a, gsem_b, ssem_a, ssem_b):
        c = lax.axis_index("c")
        s = lax.axis_index("s")

        _zero_acc(acc, ztile, s, zrows, feat)
        plsc.subcore_barrier()

        def bbody(bb, _):
            w = jnp.where(bb < nblk, s, s + NS)
            gb = w * nblk + jnp.where(bb < nblk, bb, bb - nblk)
            pltpu.sync_copy(src_hbm.at[c * (NW * nblk) + gb], sidx)
            pltpu.sync_copy(dst_hbm.at[c * (NW * nblk) + gb], didx)
            _agg_block_pipeline(g_hbm, acc, sidx, didx, rows_a, rows_b,
                                gsem_a, gsem_b, ssem_a, ssem_b)
            return 0

        lax.fori_loop(0, 2 * nblk, bbody, 0)
        plsc.subcore_barrier()

        off = 0
        while off < wrows:
            wb = min(CHUNK, wrows - off)
            r0 = s * wrows + off
            pltpu.sync_copy(acc.at[pl.ds(r0, wb), :],
                            rows_a.at[pl.ds(0, wb), :])
            pltpu.sync_copy(rows_a.at[pl.ds(0, wb), :],
                            out_hbm.at[c, pl.ds(r0, wb), :])
            off += wb

    return agg_kernel


def _mm_scale_body(x_ref, w_ref, d0_ref, d1_ref, o_ref):
    deg = d0_ref[...] + d1_ref[...] + 1.0
    dis = lax.rsqrt(deg)
    h = jnp.dot(x_ref[...], w_ref[...],
                preferred_element_type=jnp.float32,
                precision=lax.Precision.HIGHEST)
    g = dis * h
    o_ref[0, ...] = g
    o_ref[1, ...] = g


def _mm_scale(x, w, d0, d1, blk):
    npd, din = x.shape
    feat = w.shape[1]
    return pl.pallas_call(
        _mm_scale_body,
        grid=(npd // blk,),
        in_specs=[
            pl.BlockSpec((blk, din), lambda i: (i, 0)),
            pl.BlockSpec((din, feat), lambda i: (0, 0)),
            pl.BlockSpec((blk, 1), lambda i: (i, 0)),
            pl.BlockSpec((blk, 1), lambda i: (i, 0)),
        ],
        out_specs=pl.BlockSpec((2, blk, feat), lambda i: (0, i, 0)),
        out_shape=jax.ShapeDtypeStruct((2, npd, feat), jnp.float32),
    )(x, w, d0, d1)


def _mid_body(p0_ref, p1_ref, g_ref, d0_ref, d1_ref, b_ref, w_ref, o_ref):
    deg = d0_ref[...] + d1_ref[...] + 1.0
    dis = lax.rsqrt(deg)
    z = dis * (p0_ref[...] + p1_ref[...] + g_ref[...]) + b_ref[...]
    z = jnp.maximum(z, 0.0)
    h = jnp.dot(z, w_ref[...],
                preferred_element_type=jnp.float32,
                precision=lax.Precision.HIGHEST)
    g = dis * h
    o_ref[0, ...] = g
    o_ref[1, ...] = g


def _mid(p0, p1, g, d0, d1, b, w, blk):
    npd, din = g.shape
    feat = w.shape[1]
    return pl.pallas_call(
        _mid_body,
        grid=(npd // blk,),
        in_specs=[
            pl.BlockSpec((blk, din), lambda i: (i, 0)),
            pl.BlockSpec((blk, din), lambda i: (i, 0)),
            pl.BlockSpec((blk, din), lambda i: (i, 0)),
            pl.BlockSpec((blk, 1), lambda i: (i, 0)),
            pl.BlockSpec((blk, 1), lambda i: (i, 0)),
            pl.BlockSpec((din,), lambda i: (0,)),
            pl.BlockSpec((din, feat), lambda i: (0, 0)),
        ],
        out_specs=pl.BlockSpec((2, blk, feat), lambda i: (0, i, 0)),
        out_shape=jax.ShapeDtypeStruct((2, npd, feat), jnp.float32),
    )(p0, p1, g, d0, d1, b, w)


def _final_body(q_ref, g_ref, d0_ref, d1_ref, b_ref, o_ref):
    deg = d0_ref[...] + d1_ref[...] + 1.0
    dis = lax.rsqrt(deg)
    o_ref[...] = dis * (q_ref[...] + g_ref[...]) + b_ref[...]


def _final(q, g, d0, d1, b, blk):
    npd, feat = g.shape
    return pl.pallas_call(
        _final_body,
        grid=(npd // blk,),
        in_specs=[
            pl.BlockSpec((blk, feat), lambda i: (i, 0)),
            pl.BlockSpec((blk, feat), lambda i: (i, 0)),
            pl.BlockSpec((blk, 1), lambda i: (i, 0)),
            pl.BlockSpec((blk, 1), lambda i: (i, 0)),
            pl.BlockSpec((feat,), lambda i: (0,)),
        ],
        out_specs=pl.BlockSpec((blk, feat), lambda i: (i, 0)),
        out_shape=jax.ShapeDtypeStruct((npd, feat), jnp.float32),
    )(q, g, d0, d1, b)


def kernel(x, edge_index, W1, b1, W2, b2):
    n, d = x.shape
    e = edge_index.shape[1]
    npad = _pad_to(n, 1024)
    ept = _pad_to(e, NW * CHUNK * 2) // NW
    epad = ept * NW
    cpt = ept // CHUNK

    pad = epad - e
    spread = jnp.arange(pad, dtype=jnp.int32) % 128
    src = jnp.concatenate(
        [edge_index[0], jnp.zeros((pad,), jnp.int32)])
    dst = jnp.concatenate(
        [edge_index[1], n + spread])
    nblk = cpt // NB
    scoff = (jnp.arange(epad, dtype=jnp.int32) // (NS * ept)) * npad
    src3 = (src + scoff).reshape(NW * nblk, NB, CHUNK)
    srcloc = jnp.stack([src, src + npad]).reshape(NC * NW * nblk, NB, CHUNK)
    dst3 = dst.reshape(NW * nblk, NB, CHUNK)
    dst2 = dst.reshape(NW * (ept // (NB * CHUNK)), NB * CHUNK)
    xp = jnp.concatenate(
        [x, jnp.zeros((npad - n, d), jnp.float32)])

    half = npad // NC
    dloc = []
    for c in range(NC):
        dl = dst - c * half
        ok = (dl >= 0) & (dl < half)
        dloc.append(jnp.where(ok, dl, half + (dst % 128)))
    dstloc = jnp.stack(dloc).reshape(NC * NW * nblk, NB, CHUNK)

    deg_parts = _make_deg_kernel(npad, ept)(dst2)
    d0 = deg_parts[0].reshape(npad, 1)
    d1 = deg_parts[1].reshape(npad, 1)

    blk = 1024
    g1d = _mm_scale(xp, W1, d0, d1, blk)
    g1s = g1d.reshape(2 * npad, W1.shape[1])

    agg1 = _make_agg_kernel(npad, W1.shape[1], cpt)(g1s, src3, dst3)
    g2d = _mid(agg1[0], agg1[1], g1d[0], d0, d1, b1, W2, blk)
    g2s = g2d.reshape(2 * npad, W2.shape[1])

    agg2 = _make_agg_split_kernel(npad, W2.shape[1], cpt)(g2s, srcloc, dstloc)
    q = agg2.reshape(npad, W2.shape[1])
    out = _final(q, g2d[0], d0, d1, b2, blk)
    return out[:n]


# speedup vs baseline: 2.4106x; 2.4106x over previous
import functools

import jax
import jax.numpy as jnp
from jax import lax
from jax.experimental import pallas as pl
from jax.experimental.pallas import tpu as pltpu
from jax.experimental.pallas import tpu_sc as plsc

NC = 2
NS = 16
NW = NC * NS
L = 16

CHUNK = 128


def _pad_to(n, m):
    return ((n + m - 1) // m) * m


def _sc_mesh():
    return plsc.VectorSubcoreMesh(
        core_axis_name="c", subcore_axis_name="s", num_cores=NC, num_subcores=NS
    )


_SC_PARAMS = pltpu.CompilerParams(
    needs_layout_passes=False, use_tc_tiling_on_sc=False
)


def _make_deg_kernel(npad, ept):
    npass = 4
    seg = npad // npass
    cb = seg // NS
    dseg = NB * CHUNK
    dblk = ept // dseg

    @functools.partial(
        pl.kernel,
        out_type=jax.ShapeDtypeStruct((NC, npad), jnp.float32),
        mesh=_sc_mesh(),
        compiler_params=_SC_PARAMS,
        scratch_types=[
            pltpu.VMEM((npad,), jnp.float32),
            pltpu.VMEM((dseg,), jnp.int32),
            pltpu.VMEM_SHARED((NS, seg), jnp.float32),
            pltpu.VMEM((NS, cb), jnp.float32),
            pltpu.VMEM((cb,), jnp.float32),
        ],
    )
    def deg_kernel(dst_hbm, out_hbm, hist, didx, staging, colblk, summed):
        c = lax.axis_index("c")
        s = lax.axis_index("s")
        wid = c * NS + s

        zero16 = jnp.zeros((L,), jnp.float32)

        def zbody(i, _):
            hist[pl.ds(i * L, L)] = zero16
            return 0

        lax.fori_loop(0, npad // L, zbody, 0)

        ones16 = jnp.ones((L,), jnp.float32)

        def abody(i, _):
            d = didx[pl.ds(i * L, L)]
            plsc.addupdate_scatter(hist, [d], ones16)
            return 0

        def dbody(bb, _):
            pltpu.sync_copy(dst_hbm.at[wid * dblk + bb], didx)
            lax.fori_loop(0, dseg // L, abody, 0)
            return 0

        lax.fori_loop(0, dblk, dbody, 0)

        def rbody(i, _):
            v = colblk[0, pl.ds(i * L, L)]
            for t in range(1, NS):
                v = v + colblk[t, pl.ds(i * L, L)]
            summed[pl.ds(i * L, L)] = v
            return 0

        for p in range(npass):
            pltpu.sync_copy(hist.at[pl.ds(p * seg, seg)], staging.at[s])
            plsc.subcore_barrier()
            pltpu.sync_copy(staging.at[:, pl.ds(s * cb, cb)], colblk)
            lax.fori_loop(0, cb // L, rbody, 0)
            pltpu.sync_copy(
                summed, out_hbm.at[c, pl.ds(p * seg + s * cb, cb)])
            plsc.subcore_barrier()

    return deg_kernel


NB = 16


def _agg_block_pipeline(g_hbm, acc, sidx, didx, rows_a, rows_b,
                        gsem_a, gsem_b, ssem_a, ssem_b):
    def gath(j, buf, sem):
        pltpu.async_copy(g_hbm.at[sidx.at[j]], buf, sem)

    def scat(j, buf, sem):
        pltpu.async_copy(buf, acc.at[didx.at[j]], sem, add=True)

    def wait_g(buf, sem):
        pltpu.make_async_copy(g_hbm.at[sidx.at[0]], buf, sem).wait()

    def wait_s(buf, sem):
        pltpu.make_async_copy(buf, acc.at[didx.at[0]], sem).wait()

    gath(0, rows_a, gsem_a)
    gath(1, rows_b, gsem_b)
    wait_g(rows_a, gsem_a)
    scat(0, rows_a, ssem_a)
    wait_s(rows_a, ssem_a)
    gath(2, rows_a, gsem_a)
    wait_g(rows_b, gsem_b)
    scat(1, rows_b, ssem_b)

    def pbody(jj, _):
        j0 = 2 * jj
        wait_s(rows_b, ssem_b)
        gath(j0 + 1, rows_b, gsem_b)
        wait_g(rows_a, gsem_a)
        scat(j0, rows_a, ssem_a)
        wait_s(rows_a, ssem_a)
        gath(j0 + 2, rows_a, gsem_a)
        wait_g(rows_b, gsem_b)
        scat(j0 + 1, rows_b, ssem_b)
        return 0

    lax.fori_loop(1, NB // 2 - 1, pbody, 0)

    wait_s(rows_b, ssem_b)
    gath(NB - 1, rows_b, gsem_b)
    wait_g(rows_a, gsem_a)
    scat(NB - 2, rows_a, ssem_a)
    wait_g(rows_b, gsem_b)
    scat(NB - 1, rows_b, ssem_b)
    wait_s(rows_a, ssem_a)
    wait_s(rows_b, ssem_b)


def _zero_acc(acc, ztile, s, zrows, feat):
    zero16 = jnp.zeros((L,), jnp.float32)
    for i in range(L):
        for j in range(feat // L):
            ztile[i, pl.ds(j * L, L)] = zero16

    def zbody(i, _):
        pltpu.sync_copy(ztile, acc.at[pl.ds((s * zrows + i) * L, L), :])
        return 0

    lax.fori_loop(0, zrows, zbody, 0)


def _make_agg_kernel(npad, feat, cpt):
    zrows = npad // NS // L
    wb = 128
    wchunks = npad // NS // wb
    nblk = cpt // NB

    @functools.partial(
        pl.kernel,
        out_type=jax.ShapeDtypeStruct((NC, npad, feat), jnp.float32),
        mesh=_sc_mesh(),
        compiler_params=_SC_PARAMS,
        scratch_types=[
            pltpu.VMEM((NB, CHUNK), jnp.int32),
            pltpu.VMEM((NB, CHUNK), jnp.int32),
            pltpu.VMEM((CHUNK, feat), jnp.float32),
            pltpu.VMEM((CHUNK, feat), jnp.float32),
            pltpu.VMEM_SHARED((npad, feat), jnp.float32),
            pltpu.VMEM((L, feat), jnp.float32),
            pltpu.SemaphoreType.DMA,
            pltpu.SemaphoreType.DMA,
            pltpu.SemaphoreType.DMA,
            pltpu.SemaphoreType.DMA,
        ],
    )
    def agg_kernel(g_hbm, src_hbm, dst_hbm, out_hbm,
                   sidx, didx, rows_a, rows_b, acc, ztile,
                   gsem_a, gsem_b, ssem_a, ssem_b):
        c = lax.axis_index("c")
        s = lax.axis_index("s")
        wid = c * NS + s

        _zero_acc(acc, ztile, s, zrows, feat)
        plsc.subcore_barrier()

        def bbody(bb, _):
            pltpu.sync_copy(src_hbm.at[wid * nblk + bb], sidx)
            pltpu.sync_copy(dst_hbm.at[wid * nblk + bb], didx)
            _agg_block_pipeline(g_hbm, acc, sidx, didx, rows_a, rows_b,
                                gsem_a, gsem_b, ssem_a, ssem_b)
            return 0

        lax.fori_loop(0, nblk, bbody, 0)
        plsc.subcore_barrier()

        def wbody(k, _):
            r0 = (s * wchunks + k) * wb
            pltpu.sync_copy(acc.at[pl.ds(r0, wb), :], rows_a)
            pltpu.sync_copy(rows_a, out_hbm.at[c, pl.ds(r0, wb), :])
            return 0

        lax.fori_loop(0, wchunks, wbody, 0)

    return agg_kernel


def _make_agg_split_kernel(npad, feat, cpt):
    half = npad // NC
    nacc = half + 256
    zrows = nacc // NS // L
    wrows = half // NS
    nblk = cpt // NB

    @functools.partial(
        pl.kernel,
        out_type=jax.ShapeDtypeStruct((NC, half, feat), jnp.float32),
        mesh=_sc_mesh(),
        compiler_params=_SC_PARAMS,
        scratch_types=[
            pltpu.VMEM((NB, CHUNK), jnp.int32),
            pltpu.VMEM((NB, CHUNK), jnp.int32),
            pltpu.VMEM((CHUNK, feat), jnp.float32),
            pltpu.VMEM((CHUNK, feat), jnp.float32),
            pltpu.VMEM_SHARED((nacc, feat), jnp.float32),
            pltpu.VMEM((L, feat), jnp.float32),
            pltpu.SemaphoreType.DMA,
            pltpu.SemaphoreType.DMA,
            pltpu.SemaphoreType.DMA,
            pltpu.SemaphoreType.DMA,
        ],
    )
    def agg_kernel(g_hbm, src_hbm, dst_hbm, out_hbm,
                   sidx, didx, rows_a, rows_b, acc, ztile,
                   gsem_a, gsem_b, ssem_a, ssem_b):
        c = lax.axis_index("c")
        s = lax.axis_index("s")

        _zero_acc(acc, ztile, s, zrows, feat)
        plsc.subcore_barrier()

        def bbody(bb, _):
            w = jnp.where(bb < nblk, s, s + NS)
            gb = w * nblk + jnp.where(bb < nblk, bb, bb - nblk)
            pltpu.sync_copy(src_hbm.at[c * (NW * nblk) + gb], sidx)
            pltpu.sync_copy(dst_hbm.at[c * (NW * nblk) + gb], didx)
            _agg_block_pipeline(g_hbm, acc, sidx, didx, rows_a, rows_b,
                                gsem_a, gsem_b, ssem_a, ssem_b)
            return 0

        lax.fori_loop(0, 2 * nblk, bbody, 0)
        plsc.subcore_barrier()

        off = 0
        while off < wrows:
            wb = min(CHUNK, wrows - off)
            r0 = s * wrows + off
            pltpu.sync_copy(acc.at[pl.ds(r0, wb), :],
                            rows_a.at[pl.ds(0, wb), :])
            pltpu.sync_copy(rows_a.at[pl.ds(0, wb), :],
                            out_hbm.at[c, pl.ds(r0, wb), :])
            off += wb

    return agg_kernel


def _mm_scale_body(x_ref, w_ref, d0_ref, d1_ref, o_ref):
    deg = d0_ref[...] + d1_ref[...] + 1.0
    dis = lax.rsqrt(deg)
    h = jnp.dot(x_ref[...], w_ref[...],
                preferred_element_type=jnp.float32,
                precision=lax.Precision.HIGHEST)
    g = dis * h
    o_ref[0, ...] = g
    o_ref[1, ...] = g


def _mm_scale(x, w, d0, d1, blk):
    npd, din = x.shape
    feat = w.shape[1]
    return pl.pallas_call(
        _mm_scale_body,
        grid=(npd // blk,),
        in_specs=[
            pl.BlockSpec((blk, din), lambda i: (i, 0)),
            pl.BlockSpec((din, feat), lambda i: (0, 0)),
            pl.BlockSpec((blk, 1), lambda i: (i, 0)),
            pl.BlockSpec((blk, 1), lambda i: (i, 0)),
        ],
        out_specs=pl.BlockSpec((2, blk, feat), lambda i: (0, i, 0)),
        out_shape=jax.ShapeDtypeStruct((2, npd, feat), jnp.float32),
    )(x, w, d0, d1)


def _mid_body(p0_ref, p1_ref, g_ref, d0_ref, d1_ref, b_ref, w_ref, o_ref):
    deg = d0_ref[...] + d1_ref[...] + 1.0
    dis = lax.rsqrt(deg)
    z = dis * (p0_ref[...] + p1_ref[...] + g_ref[...]) + b_ref[...]
    z = jnp.maximum(z, 0.0)
    h = jnp.dot(z, w_ref[...],
                preferred_element_type=jnp.float32,
                precision=lax.Precision.HIGHEST)
    g = dis * h
    o_ref[0, ...] = g
    o_ref[1, ...] = g


def _mid(p0, p1, g, d0, d1, b, w, blk):
    npd, din = g.shape
    feat = w.shape[1]
    return pl.pallas_call(
        _mid_body,
        grid=(npd // blk,),
        in_specs=[
            pl.BlockSpec((blk, din), lambda i: (i, 0)),
            pl.BlockSpec((blk, din), lambda i: (i, 0)),
            pl.BlockSpec((blk, din), lambda i: (i, 0)),
            pl.BlockSpec((blk, 1), lambda i: (i, 0)),
            pl.BlockSpec((blk, 1), lambda i: (i, 0)),
            pl.BlockSpec((din,), lambda i: (0,)),
            pl.BlockSpec((din, feat), lambda i: (0, 0)),
        ],
        out_specs=pl.BlockSpec((2, blk, feat), lambda i: (0, i, 0)),
        out_shape=jax.ShapeDtypeStruct((2, npd, feat), jnp.float32),
    )(p0, p1, g, d0, d1, b, w)


def _final_body(q_ref, g_ref, d0_ref, d1_ref, b_ref, o_ref):
    deg = d0_ref[...] + d1_ref[...] + 1.0
    dis = lax.rsqrt(deg)
    o_ref[...] = dis * (q_ref[...] + g_ref[...]) + b_ref[...]


def _final(q, g, d0, d1, b, blk):
    npd, feat = g.shape
    return pl.pallas_call(
        _final_body,
        grid=(npd // blk,),
        in_specs=[
            pl.BlockSpec((blk, feat), lambda i: (i, 0)),
            pl.BlockSpec((blk, feat), lambda i: (i, 0)),
            pl.BlockSpec((blk, 1), lambda i: (i, 0)),
            pl.BlockSpec((blk, 1), lambda i: (i, 0)),
            pl.BlockSpec((feat,), lambda i: (0,)),
        ],
        out_specs=pl.BlockSpec((blk, feat), lambda i: (i, 0)),
        out_shape=jax.ShapeDtypeStruct((npd, feat), jnp.float32),
    )(q, g, d0, d1, b)


def kernel(x, edge_index, W1, b1, W2, b2):
    n, d = x.shape
    e = edge_index.shape[1]
    npad = _pad_to(n, 1024)
    ept = _pad_to(e, NW * CHUNK * 2) // NW
    epad = ept * NW
    cpt = ept // CHUNK

    pad = epad - e
    spread = jnp.arange(pad, dtype=jnp.int32) % 128
    src = jnp.concatenate(
        [edge_index[0], (jnp.arange(pad, dtype=jnp.int32) * 53) % n])
    dst = jnp.concatenate(
        [edge_index[1], n + spread])
    nblk = cpt // NB
    scoff = (jnp.arange(epad, dtype=jnp.int32) // (NS * ept)) * npad
    src3 = (src + scoff).reshape(NW * nblk, NB, CHUNK)
    srcloc = jnp.stack([src, src + npad]).reshape(NC * NW * nblk, NB, CHUNK)
    dst3 = dst.reshape(NW * nblk, NB, CHUNK)
    dst2 = dst.reshape(NW * (ept // (NB * CHUNK)), NB * CHUNK)
    xp = jnp.concatenate(
        [x, jnp.zeros((npad - n, d), jnp.float32)])

    half = npad // NC
    dloc = []
    for c in range(NC):
        dl = dst - c * half
        ok = (dl >= 0) & (dl < half)
        dloc.append(jnp.where(ok, dl, half + (dst % 128)))
    dstloc = jnp.stack(dloc).reshape(NC * NW * nblk, NB, CHUNK)

    deg_parts = _make_deg_kernel(npad, ept)(dst2)
    d0 = deg_parts[0].reshape(npad, 1)
    d1 = deg_parts[1].reshape(npad, 1)

    blk = 1024
    g1d = _mm_scale(xp, W1, d0, d1, blk)
    g1s = g1d.reshape(2 * npad, W1.shape[1])

    agg1 = _make_agg_kernel(npad, W1.shape[1], cpt)(g1s, src3, dst3)
    g2d = _mid(agg1[0], agg1[1], g1d[0], d0, d1, b1, W2, blk)
    g2s = g2d.reshape(2 * npad, W2.shape[1])

    agg2 = _make_agg_split_kernel(npad, W2.shape[1], cpt)(g2s, srcloc, dstloc)
    q = agg2.reshape(npad, W2.shape[1])
    out = _final(q, g2d[0], d0, d1, b2, blk)
    return out[:n]


# speedup vs baseline: 2.4697x; 1.0245x over previous
import functools

import jax
import jax.numpy as jnp
from jax import lax
from jax.experimental import pallas as pl
from jax.experimental.pallas import tpu as pltpu
from jax.experimental.pallas import tpu_sc as plsc

NC = 2
NS = 16
NW = NC * NS
L = 16

CHUNK = 128


def _pad_to(n, m):
    return ((n + m - 1) // m) * m


def _sc_mesh():
    return plsc.VectorSubcoreMesh(
        core_axis_name="c", subcore_axis_name="s", num_cores=NC, num_subcores=NS
    )


_SC_PARAMS = pltpu.CompilerParams(
    needs_layout_passes=False, use_tc_tiling_on_sc=False
)


def _make_deg_kernel(npad, ept):
    npass = 4
    seg = npad // npass
    cb = seg // NS
    dseg = NB * CHUNK
    dblk = ept // dseg

    @functools.partial(
        pl.kernel,
        out_type=jax.ShapeDtypeStruct((NC, npad), jnp.float32),
        mesh=_sc_mesh(),
        compiler_params=_SC_PARAMS,
        scratch_types=[
            pltpu.VMEM((npad,), jnp.float32),
            pltpu.VMEM((dseg,), jnp.int32),
            pltpu.VMEM_SHARED((NS, seg), jnp.float32),
            pltpu.VMEM((NS, cb), jnp.float32),
            pltpu.VMEM((cb,), jnp.float32),
        ],
    )
    def deg_kernel(dst_hbm, out_hbm, hist, didx, staging, colblk, summed):
        c = lax.axis_index("c")
        s = lax.axis_index("s")
        wid = c * NS + s

        zero16 = jnp.zeros((L,), jnp.float32)

        def zbody(i, _):
            hist[pl.ds(i * L, L)] = zero16
            return 0

        lax.fori_loop(0, npad // L, zbody, 0)

        ones16 = jnp.ones((L,), jnp.float32)

        def abody(i, _):
            d = didx[pl.ds(i * L, L)]
            plsc.addupdate_scatter(hist, [d], ones16)
            return 0

        def dbody(bb, _):
            pltpu.sync_copy(dst_hbm.at[wid * dblk + bb], didx)
            lax.fori_loop(0, dseg // L, abody, 0)
            return 0

        lax.fori_loop(0, dblk, dbody, 0)

        def rbody(i, _):
            v = colblk[0, pl.ds(i * L, L)]
            for t in range(1, NS):
                v = v + colblk[t, pl.ds(i * L, L)]
            summed[pl.ds(i * L, L)] = v
            return 0

        for p in range(npass):
            pltpu.sync_copy(hist.at[pl.ds(p * seg, seg)], staging.at[s])
            plsc.subcore_barrier()
            pltpu.sync_copy(staging.at[:, pl.ds(s * cb, cb)], colblk)
            lax.fori_loop(0, cb // L, rbody, 0)
            pltpu.sync_copy(
                summed, out_hbm.at[c, pl.ds(p * seg + s * cb, cb)])
            plsc.subcore_barrier()

    return deg_kernel


NB = 16


def _agg_block_pipeline(g_hbm, acc, sidx, didx, rows_a, rows_b,
                        gsem_a, gsem_b, ssem_a, ssem_b):
    def gath(j, buf, sem):
        pltpu.async_copy(g_hbm.at[sidx.at[j]], buf, sem)

    def scat(j, buf, sem):
        pltpu.async_copy(buf, acc.at[didx.at[j]], sem, add=True)

    def wait_g(buf, sem):
        pltpu.make_async_copy(g_hbm.at[sidx.at[0]], buf, sem).wait()

    def wait_s(buf, sem):
        pltpu.make_async_copy(buf, acc.at[didx.at[0]], sem).wait()

    gath(0, rows_a, gsem_a)
    gath(1, rows_b, gsem_b)
    wait_g(rows_a, gsem_a)
    scat(0, rows_a, ssem_a)
    wait_s(rows_a, ssem_a)
    gath(2, rows_a, gsem_a)
    wait_g(rows_b, gsem_b)
    scat(1, rows_b, ssem_b)

    def pbody(jj, _):
        j0 = 2 * jj
        wait_s(rows_b, ssem_b)
        gath(j0 + 1, rows_b, gsem_b)
        wait_g(rows_a, gsem_a)
        scat(j0, rows_a, ssem_a)
        wait_s(rows_a, ssem_a)
        gath(j0 + 2, rows_a, gsem_a)
        wait_g(rows_b, gsem_b)
        scat(j0 + 1, rows_b, ssem_b)
        return 0

    lax.fori_loop(1, NB // 2 - 1, pbody, 0)

    wait_s(rows_b, ssem_b)
    gath(NB - 1, rows_b, gsem_b)
    wait_g(rows_a, gsem_a)
    scat(NB - 2, rows_a, ssem_a)
    wait_g(rows_b, gsem_b)
    scat(NB - 1, rows_b, ssem_b)
    wait_s(rows_a, ssem_a)
    wait_s(rows_b, ssem_b)


def _zero_acc(acc, ztile, s, zrows, feat):
    zero16 = jnp.zeros((L,), jnp.float32)
    for i in range(L):
        for j in range(feat // L):
            ztile[i, pl.ds(j * L, L)] = zero16

    def zbody(i, _):
        pltpu.sync_copy(ztile, acc.at[pl.ds((s * zrows + i) * L, L), :])
        return 0

    lax.fori_loop(0, zrows, zbody, 0)


def _make_agg_kernel(npad, feat, cpt):
    zrows = npad // NS // L
    wb = 128
    wchunks = npad // NS // wb
    nblk = cpt // NB

    @functools.partial(
        pl.kernel,
        out_type=jax.ShapeDtypeStruct((NC, npad, feat), jnp.float32),
        mesh=_sc_mesh(),
        compiler_params=_SC_PARAMS,
        scratch_types=[
            pltpu.VMEM((NB, CHUNK), jnp.int32),
            pltpu.VMEM((NB, CHUNK), jnp.int32),
            pltpu.VMEM((CHUNK, feat), jnp.float32),
            pltpu.VMEM((CHUNK, feat), jnp.float32),
            pltpu.VMEM_SHARED((npad, feat), jnp.float32),
            pltpu.VMEM((L, feat), jnp.float32),
            pltpu.SemaphoreType.DMA,
            pltpu.SemaphoreType.DMA,
            pltpu.SemaphoreType.DMA,
            pltpu.SemaphoreType.DMA,
        ],
    )
    def agg_kernel(g_hbm, src_hbm, dst_hbm, out_hbm,
                   sidx, didx, rows_a, rows_b, acc, ztile,
                   gsem_a, gsem_b, ssem_a, ssem_b):
        c = lax.axis_index("c")
        s = lax.axis_index("s")
        wid = c * NS + s

        _zero_acc(acc, ztile, s, zrows, feat)
        plsc.subcore_barrier()

        def bbody(bb, _):
            pltpu.sync_copy(src_hbm.at[wid * nblk + bb], sidx)
            pltpu.sync_copy(dst_hbm.at[wid * nblk + bb], didx)
            _agg_block_pipeline(g_hbm, acc, sidx, didx, rows_a, rows_b,
                                gsem_a, gsem_b, ssem_a, ssem_b)
            return 0

        lax.fori_loop(0, nblk, bbody, 0)
        plsc.subcore_barrier()

        def wbody(k, _):
            r0 = (s * wchunks + k) * wb
            pltpu.sync_copy(acc.at[pl.ds(r0, wb), :], rows_a)
            pltpu.sync_copy(rows_a, out_hbm.at[c, pl.ds(r0, wb), :])
            return 0

        lax.fori_loop(0, wchunks, wbody, 0)

    return agg_kernel


def _make_agg_split_kernel(npad, feat, cpt):
    half = npad // NC
    nacc = half + 256
    zrows = nacc // NS // L
    wrows = half // NS
    nblk = cpt // NB

    @functools.partial(
        pl.kernel,
        out_type=jax.ShapeDtypeStruct((NC, half, feat), jnp.float32),
        mesh=_sc_mesh(),
        compiler_params=_SC_PARAMS,
        scratch_types=[
            pltpu.VMEM((NB, CHUNK), jnp.int32),
            pltpu.VMEM((NB, CHUNK), jnp.int32),
            pltpu.VMEM((CHUNK, feat), jnp.float32),
            pltpu.VMEM((CHUNK, feat), jnp.float32),
            pltpu.VMEM_SHARED((nacc, feat), jnp.float32),
            pltpu.VMEM((L, feat), jnp.float32),
            pltpu.SemaphoreType.DMA,
            pltpu.SemaphoreType.DMA,
            pltpu.SemaphoreType.DMA,
            pltpu.SemaphoreType.DMA,
        ],
    )
    def agg_kernel(g_hbm, src_hbm, dst_hbm, out_hbm,
                   sidx, didx, rows_a, rows_b, acc, ztile,
                   gsem_a, gsem_b, ssem_a, ssem_b):
        c = lax.axis_index("c")
        s = lax.axis_index("s")

        _zero_acc(acc, ztile, s, zrows, feat)
        plsc.subcore_barrier()

        def bbody(bb, _):
            w = jnp.where(bb < nblk, s, s + NS)
            gb = w * nblk + jnp.where(bb < nblk, bb, bb - nblk)
            pltpu.sync_copy(src_hbm.at[c * (NW * nblk) + gb], sidx)
            pltpu.sync_copy(dst_hbm.at[c * (NW * nblk) + gb], didx)
            _agg_block_pipeline(g_hbm, acc, sidx, didx, rows_a, rows_b,
                                gsem_a, gsem_b, ssem_a, ssem_b)
            return 0

        lax.fori_loop(0, 2 * nblk, bbody, 0)
        plsc.subcore_barrier()

        off = 0
        while off < wrows:
            wb = min(CHUNK, wrows - off)
            r0 = s * wrows + off
            pltpu.sync_copy(acc.at[pl.ds(r0, wb), :],
                            rows_a.at[pl.ds(0, wb), :])
            pltpu.sync_copy(rows_a.at[pl.ds(0, wb), :],
                            out_hbm.at[c, pl.ds(r0, wb), :])
            off += wb

    return agg_kernel


def _make_prep(e, epad, n, npad):
    ec = e // CHUNK
    ep = epad // CHUNK
    hb = ep // 2
    half = npad // NC

    def body(ei_ref, sa_ref, sl_ref, dp_ref, dl_ref):
        s = ei_ref[0]
        d = ei_ref[1]
        rowi = lax.broadcasted_iota(jnp.int32, (ep - ec, CHUNK), 0)
        lane = lax.broadcasted_iota(jnp.int32, (ep - ec, CHUNK), 1)
        padsrc = ((rowi * CHUNK + lane) * 53) % n
        paddst = n + lane
        sfull = jnp.concatenate([s, padsrc], axis=0)
        dfull = jnp.concatenate([d, paddst], axis=0)
        sa_ref[0:hb] = sfull[0:hb]
        sa_ref[hb:ep] = sfull[hb:ep] + npad
        sl_ref[0, ...] = sfull
        sl_ref[1, ...] = sfull + npad
        dp_ref[...] = dfull
        for c in range(NC):
            dl = dfull - c * half
            ok = (dl >= 0) & (dl < half)
            dl_ref[c, ...] = jnp.where(ok, dl, half + (dfull % 128))

    return pl.pallas_call(
        body,
        out_shape=(
            jax.ShapeDtypeStruct((ep, CHUNK), jnp.int32),
            jax.ShapeDtypeStruct((NC, ep, CHUNK), jnp.int32),
            jax.ShapeDtypeStruct((ep, CHUNK), jnp.int32),
            jax.ShapeDtypeStruct((NC, ep, CHUNK), jnp.int32),
        ),
    )


def _mm_scale_body(x_ref, w_ref, d0_ref, d1_ref, o_ref):
    n = x_ref.shape[0]
    npd = d0_ref.shape[0]
    feat = w_ref.shape[1]
    deg = d0_ref[...] + d1_ref[...] + 1.0
    dis = lax.rsqrt(deg)
    h = jnp.dot(x_ref[...], w_ref[...],
                preferred_element_type=jnp.float32,
                precision=lax.Precision.HIGHEST)
    g = dis[0:n] * h
    o_ref[0, 0:n] = g
    o_ref[1, 0:n] = g
    z = jnp.zeros((npd - n, feat), jnp.float32)
    o_ref[0, n:npd] = z
    o_ref[1, n:npd] = z


def _mm_scale(x, w, d0, d1, npad):
    feat = w.shape[1]
    return pl.pallas_call(
        _mm_scale_body,
        out_shape=jax.ShapeDtypeStruct((2, npad, feat), jnp.float32),
    )(x, w, d0, d1)


def _mid_body(p0_ref, p1_ref, g_ref, d0_ref, d1_ref, b_ref, w_ref, o_ref):
    deg = d0_ref[...] + d1_ref[...] + 1.0
    dis = lax.rsqrt(deg)
    z = dis * (p0_ref[...] + p1_ref[...] + g_ref[...]) + b_ref[...]
    z = jnp.maximum(z, 0.0)
    h = jnp.dot(z, w_ref[...],
                preferred_element_type=jnp.float32,
                precision=lax.Precision.HIGHEST)
    g = dis * h
    o_ref[0, ...] = g
    o_ref[1, ...] = g


def _mid(p0, p1, g, d0, d1, b, w, blk):
    npd, din = g.shape
    feat = w.shape[1]
    return pl.pallas_call(
        _mid_body,
        grid=(npd // blk,),
        in_specs=[
            pl.BlockSpec((blk, din), lambda i: (i, 0)),
            pl.BlockSpec((blk, din), lambda i: (i, 0)),
            pl.BlockSpec((blk, din), lambda i: (i, 0)),
            pl.BlockSpec((blk, 1), lambda i: (i, 0)),
            pl.BlockSpec((blk, 1), lambda i: (i, 0)),
            pl.BlockSpec((din,), lambda i: (0,)),
            pl.BlockSpec((din, feat), lambda i: (0, 0)),
        ],
        out_specs=pl.BlockSpec((2, blk, feat), lambda i: (0, i, 0)),
        out_shape=jax.ShapeDtypeStruct((2, npd, feat), jnp.float32),
    )(p0, p1, g, d0, d1, b, w)


def _final_body(q_ref, g_ref, d0_ref, d1_ref, b_ref, o_ref):
    deg = d0_ref[...] + d1_ref[...] + 1.0
    dis = lax.rsqrt(deg)
    o_ref[...] = dis * (q_ref[0, ...] + g_ref[...]) + b_ref[...]


def _final(q, g, d0, d1, b, blk):
    npd, feat = g.shape
    hblk = (npd // NC) // blk

    return pl.pallas_call(
        _final_body,
        grid=(npd // blk,),
        in_specs=[
            pl.BlockSpec((1, blk, feat),
                         lambda i: (i // hblk, i % hblk, 0)),
            pl.BlockSpec((blk, feat), lambda i: (i, 0)),
            pl.BlockSpec((blk, 1), lambda i: (i, 0)),
            pl.BlockSpec((blk, 1), lambda i: (i, 0)),
            pl.BlockSpec((feat,), lambda i: (0,)),
        ],
        out_specs=pl.BlockSpec((blk, feat), lambda i: (i, 0)),
        out_shape=jax.ShapeDtypeStruct((npd, feat), jnp.float32),
    )(q, g, d0, d1, b)


def kernel(x, edge_index, W1, b1, W2, b2):
    n, d = x.shape
    e = edge_index.shape[1]
    npad = _pad_to(n, 1024)
    ept = _pad_to(e, NW * CHUNK * 2) // NW
    epad = ept * NW
    cpt = ept // CHUNK

    ei = edge_index
    if e % CHUNK:
        extra = CHUNK - e % CHUNK
        ei = jnp.concatenate(
            [ei, jnp.stack([(jnp.arange(extra, dtype=jnp.int32) * 53) % n,
                            jnp.full((extra,), n, jnp.int32)])], axis=1)
        e = e + extra
    ei3 = ei.reshape(2, e // CHUNK, CHUNK)

    nblk = cpt // NB
    sa, sl, dp, dl = _make_prep(e, epad, n, npad)(ei3)
    src3 = sa.reshape(NW * nblk, NB, CHUNK)
    srcloc = sl.reshape(NC * NW * nblk, NB, CHUNK)
    dst3 = dp.reshape(NW * nblk, NB, CHUNK)
    dst2 = dp.reshape(NW * (ept // (NB * CHUNK)), NB * CHUNK)
    dstloc = dl.reshape(NC * NW * nblk, NB, CHUNK)

    deg_parts = _make_deg_kernel(npad, ept)(dst2)
    d0 = deg_parts[0].reshape(npad, 1)
    d1 = deg_parts[1].reshape(npad, 1)

    blk = 1024
    g1d = _mm_scale(x, W1, d0, d1, npad)
    g1s = g1d.reshape(2 * npad, W1.shape[1])

    agg1 = _make_agg_kernel(npad, W1.shape[1], cpt)(g1s, src3, dst3)
    g2d = _mid(agg1[0], agg1[1], g1d[0], d0, d1, b1, W2, blk)
    g2s = g2d.reshape(2 * npad, W2.shape[1])

    agg2 = _make_agg_split_kernel(npad, W2.shape[1], cpt)(g2s, srcloc, dstloc)
    out = _final(agg2, g2d[0], d0, d1, b2, blk)
    return out[:n]


# speedup vs baseline: 2.5209x; 1.0207x over previous
import functools

import jax
import jax.numpy as jnp
from jax import lax
from jax.experimental import pallas as pl
from jax.experimental.pallas import tpu as pltpu
from jax.experimental.pallas import tpu_sc as plsc

NC = 2
NS = 16
NW = NC * NS
L = 16

CHUNK = 128


def _pad_to(n, m):
    return ((n + m - 1) // m) * m


def _sc_mesh():
    return plsc.VectorSubcoreMesh(
        core_axis_name="c", subcore_axis_name="s", num_cores=NC, num_subcores=NS
    )


_SC_PARAMS = pltpu.CompilerParams(
    needs_layout_passes=False, use_tc_tiling_on_sc=False
)


def _make_deg_kernel(npad, ept):
    npass = 4
    seg = npad // npass
    cb = seg // NS
    dseg = NB * CHUNK
    dblk = ept // dseg

    @functools.partial(
        pl.kernel,
        out_type=jax.ShapeDtypeStruct((NC, npad), jnp.float32),
        mesh=_sc_mesh(),
        compiler_params=_SC_PARAMS,
        scratch_types=[
            pltpu.VMEM((npad,), jnp.float32),
            pltpu.VMEM((dseg,), jnp.int32),
            pltpu.VMEM_SHARED((NS, seg), jnp.float32),
            pltpu.VMEM((NS, cb), jnp.float32),
            pltpu.VMEM((cb,), jnp.float32),
        ],
    )
    def deg_kernel(dst_hbm, out_hbm, hist, didx, staging, colblk, summed):
        c = lax.axis_index("c")
        s = lax.axis_index("s")
        wid = c * NS + s

        zero16 = jnp.zeros((L,), jnp.float32)

        def zbody(i, _):
            for u in range(4):
                hist[pl.ds((4 * i + u) * L, L)] = zero16
            return 0

        lax.fori_loop(0, npad // L // 4, zbody, 0)

        ones16 = jnp.ones((L,), jnp.float32)

        def abody(i, _):
            for u in range(4):
                d = didx[pl.ds((4 * i + u) * L, L)]
                plsc.addupdate_scatter(hist, [d], ones16)
            return 0

        def dbody(bb, _):
            pltpu.sync_copy(dst_hbm.at[wid * dblk + bb], didx)
            lax.fori_loop(0, dseg // L // 4, abody, 0)
            return 0

        lax.fori_loop(0, dblk, dbody, 0)

        def rbody(i, _):
            v = colblk[0, pl.ds(i * L, L)]
            for t in range(1, NS):
                v = v + colblk[t, pl.ds(i * L, L)]
            summed[pl.ds(i * L, L)] = v
            return 0

        for p in range(npass):
            pltpu.sync_copy(hist.at[pl.ds(p * seg, seg)], staging.at[s])
            plsc.subcore_barrier()
            pltpu.sync_copy(staging.at[:, pl.ds(s * cb, cb)], colblk)
            lax.fori_loop(0, cb // L, rbody, 0)
            pltpu.sync_copy(
                summed, out_hbm.at[c, pl.ds(p * seg + s * cb, cb)])
            plsc.subcore_barrier()

    return deg_kernel


NB = 16


def _agg_block_pipeline(g_hbm, acc, sidx, didx, rows_a, rows_b,
                        gsem_a, gsem_b, ssem_a, ssem_b):
    def gath(j, buf, sem):
        pltpu.async_copy(g_hbm.at[sidx.at[j]], buf, sem)

    def scat(j, buf, sem):
        pltpu.async_copy(buf, acc.at[didx.at[j]], sem, add=True)

    def wait_g(buf, sem):
        pltpu.make_async_copy(g_hbm.at[sidx.at[0]], buf, sem).wait()

    def wait_s(buf, sem):
        pltpu.make_async_copy(buf, acc.at[didx.at[0]], sem).wait()

    gath(0, rows_a, gsem_a)
    gath(1, rows_b, gsem_b)
    wait_g(rows_a, gsem_a)
    scat(0, rows_a, ssem_a)
    wait_s(rows_a, ssem_a)
    gath(2, rows_a, gsem_a)
    wait_g(rows_b, gsem_b)
    scat(1, rows_b, ssem_b)

    def pbody(jj, _):
        j0 = 2 * jj
        wait_s(rows_b, ssem_b)
        gath(j0 + 1, rows_b, gsem_b)
        wait_g(rows_a, gsem_a)
        scat(j0, rows_a, ssem_a)
        wait_s(rows_a, ssem_a)
        gath(j0 + 2, rows_a, gsem_a)
        wait_g(rows_b, gsem_b)
        scat(j0 + 1, rows_b, ssem_b)
        return 0

    lax.fori_loop(1, NB // 2 - 1, pbody, 0)

    wait_s(rows_b, ssem_b)
    gath(NB - 1, rows_b, gsem_b)
    wait_g(rows_a, gsem_a)
    scat(NB - 2, rows_a, ssem_a)
    wait_g(rows_b, gsem_b)
    scat(NB - 1, rows_b, ssem_b)
    wait_s(rows_a, ssem_a)
    wait_s(rows_b, ssem_b)


def _zero_acc(acc, ztile, s, zrows, feat):
    zero16 = jnp.zeros((L,), jnp.float32)
    for i in range(L):
        for j in range(feat // L):
            ztile[i, pl.ds(j * L, L)] = zero16

    def zbody(i, _):
        pltpu.sync_copy(ztile, acc.at[pl.ds((s * zrows + i) * L, L), :])
        return 0

    lax.fori_loop(0, zrows, zbody, 0)


def _make_agg_kernel(npad, feat, cpt):
    zrows = npad // NS // L
    wb = 128
    wchunks = npad // NS // wb
    nblk = cpt // NB

    @functools.partial(
        pl.kernel,
        out_type=jax.ShapeDtypeStruct((NC, npad, feat), jnp.float32),
        mesh=_sc_mesh(),
        compiler_params=_SC_PARAMS,
        scratch_types=[
            pltpu.VMEM((NB, CHUNK), jnp.int32),
            pltpu.VMEM((NB, CHUNK), jnp.int32),
            pltpu.VMEM((CHUNK, feat), jnp.float32),
            pltpu.VMEM((CHUNK, feat), jnp.float32),
            pltpu.VMEM_SHARED((npad, feat), jnp.float32),
            pltpu.VMEM((L, feat), jnp.float32),
            pltpu.SemaphoreType.DMA,
            pltpu.SemaphoreType.DMA,
            pltpu.SemaphoreType.DMA,
            pltpu.SemaphoreType.DMA,
        ],
    )
    def agg_kernel(g_hbm, src_hbm, dst_hbm, out_hbm,
                   sidx, didx, rows_a, rows_b, acc, ztile,
                   gsem_a, gsem_b, ssem_a, ssem_b):
        c = lax.axis_index("c")
        s = lax.axis_index("s")
        wid = c * NS + s

        _zero_acc(acc, ztile, s, zrows, feat)
        plsc.subcore_barrier()

        def bbody(bb, _):
            pltpu.sync_copy(src_hbm.at[wid * nblk + bb], sidx)
            pltpu.sync_copy(dst_hbm.at[wid * nblk + bb], didx)
            _agg_block_pipeline(g_hbm, acc, sidx, didx, rows_a, rows_b,
                                gsem_a, gsem_b, ssem_a, ssem_b)
            return 0

        lax.fori_loop(0, nblk, bbody, 0)
        plsc.subcore_barrier()

        def wbody(k, _):
            r0 = (s * wchunks + k) * wb
            pltpu.sync_copy(acc.at[pl.ds(r0, wb), :], rows_a)
            pltpu.sync_copy(rows_a, out_hbm.at[c, pl.ds(r0, wb), :])
            return 0

        lax.fori_loop(0, wchunks, wbody, 0)

    return agg_kernel


def _make_agg_split_kernel(npad, feat, cpt):
    half = npad // NC
    nacc = half + 256
    zrows = nacc // NS // L
    wrows = half // NS
    nblk = cpt // NB

    @functools.partial(
        pl.kernel,
        out_type=jax.ShapeDtypeStruct((NC, half, feat), jnp.float32),
        mesh=_sc_mesh(),
        compiler_params=_SC_PARAMS,
        scratch_types=[
            pltpu.VMEM((NB, CHUNK), jnp.int32),
            pltpu.VMEM((NB, CHUNK), jnp.int32),
            pltpu.VMEM((CHUNK, feat), jnp.float32),
            pltpu.VMEM((CHUNK, feat), jnp.float32),
            pltpu.VMEM_SHARED((nacc, feat), jnp.float32),
            pltpu.VMEM((L, feat), jnp.float32),
            pltpu.SemaphoreType.DMA,
            pltpu.SemaphoreType.DMA,
            pltpu.SemaphoreType.DMA,
            pltpu.SemaphoreType.DMA,
        ],
    )
    def agg_kernel(g_hbm, src_hbm, dst_hbm, out_hbm,
                   sidx, didx, rows_a, rows_b, acc, ztile,
                   gsem_a, gsem_b, ssem_a, ssem_b):
        c = lax.axis_index("c")
        s = lax.axis_index("s")

        _zero_acc(acc, ztile, s, zrows, feat)
        plsc.subcore_barrier()

        def bbody(bb, _):
            w = jnp.where(bb < nblk, s, s + NS)
            gb = w * nblk + jnp.where(bb < nblk, bb, bb - nblk)
            pltpu.sync_copy(src_hbm.at[c * (NW * nblk) + gb], sidx)
            pltpu.sync_copy(dst_hbm.at[c * (NW * nblk) + gb], didx)
            _agg_block_pipeline(g_hbm, acc, sidx, didx, rows_a, rows_b,
                                gsem_a, gsem_b, ssem_a, ssem_b)
            return 0

        lax.fori_loop(0, 2 * nblk, bbody, 0)
        plsc.subcore_barrier()

        off = 0
        while off < wrows:
            wb = min(CHUNK, wrows - off)
            r0 = s * wrows + off
            pltpu.sync_copy(acc.at[pl.ds(r0, wb), :],
                            rows_a.at[pl.ds(0, wb), :])
            pltpu.sync_copy(rows_a.at[pl.ds(0, wb), :],
                            out_hbm.at[c, pl.ds(r0, wb), :])
            off += wb

    return agg_kernel


def _make_prep(e, epad, n, npad):
    ec = e // CHUNK
    ep = epad // CHUNK
    hb = ep // 2
    half = npad // NC

    epe = ep * CHUNK
    ece = ec * CHUNK

    def body(ei_ref, sa_ref, sl_ref, dp_ref, dl_ref):
        s = ei_ref[0]
        d = ei_ref[1]
        j = lax.iota(jnp.int32, epe - ece)
        padsrc = (j * 53) % n
        paddst = n + j % 128
        sfull = jnp.concatenate([s, padsrc], axis=0)
        dfull = jnp.concatenate([d, paddst], axis=0)
        he = hb * CHUNK
        sa_ref[0:he] = sfull[0:he]
        sa_ref[he:epe] = sfull[he:epe] + npad
        sl_ref[0, ...] = sfull
        sl_ref[1, ...] = sfull + npad
        dp_ref[...] = dfull
        for c in range(NC):
            dl = dfull - c * half
            ok = (dl >= 0) & (dl < half)
            dl_ref[c, ...] = jnp.where(ok, dl, half + (dfull % 128))

    return pl.pallas_call(
        body,
        out_shape=(
            jax.ShapeDtypeStruct((epe,), jnp.int32),
            jax.ShapeDtypeStruct((NC, epe), jnp.int32),
            jax.ShapeDtypeStruct((epe,), jnp.int32),
            jax.ShapeDtypeStruct((NC, epe), jnp.int32),
        ),
    )


def _mm_scale_body(x_ref, w_ref, d0_ref, d1_ref, o_ref):
    n = x_ref.shape[0]
    npd = d0_ref.shape[0]
    feat = w_ref.shape[1]
    deg = d0_ref[...] + d1_ref[...] + 1.0
    dis = lax.rsqrt(deg)
    h = jnp.dot(x_ref[...], w_ref[...],
                preferred_element_type=jnp.float32,
                precision=lax.Precision.HIGHEST)
    g = dis[0:n] * h
    o_ref[0, 0:n] = g
    o_ref[1, 0:n] = g
    z = jnp.zeros((npd - n, feat), jnp.float32)
    o_ref[0, n:npd] = z
    o_ref[1, n:npd] = z


def _mm_scale(x, w, d0, d1, npad):
    feat = w.shape[1]
    return pl.pallas_call(
        _mm_scale_body,
        out_shape=jax.ShapeDtypeStruct((2, npad, feat), jnp.float32),
    )(x, w, d0, d1)


def _mid_body(p0_ref, p1_ref, g_ref, d0_ref, d1_ref, b_ref, w_ref, o_ref):
    deg = d0_ref[...] + d1_ref[...] + 1.0
    dis = lax.rsqrt(deg)
    z = dis * (p0_ref[...] + p1_ref[...] + g_ref[...]) + b_ref[...]
    z = jnp.maximum(z, 0.0)
    h = jnp.dot(z, w_ref[...],
                preferred_element_type=jnp.float32,
                precision=lax.Precision.HIGHEST)
    g = dis * h
    o_ref[0, ...] = g
    o_ref[1, ...] = g


def _mid(p0, p1, g, d0, d1, b, w, blk):
    npd, din = g.shape
    feat = w.shape[1]
    return pl.pallas_call(
        _mid_body,
        grid=(npd // blk,),
        in_specs=[
            pl.BlockSpec((blk, din), lambda i: (i, 0)),
            pl.BlockSpec((blk, din), lambda i: (i, 0)),
            pl.BlockSpec((blk, din), lambda i: (i, 0)),
            pl.BlockSpec((blk, 1), lambda i: (i, 0)),
            pl.BlockSpec((blk, 1), lambda i: (i, 0)),
            pl.BlockSpec((din,), lambda i: (0,)),
            pl.BlockSpec((din, feat), lambda i: (0, 0)),
        ],
        out_specs=pl.BlockSpec((2, blk, feat), lambda i: (0, i, 0)),
        out_shape=jax.ShapeDtypeStruct((2, npd, feat), jnp.float32),
    )(p0, p1, g, d0, d1, b, w)


def _final_body(q_ref, g_ref, d0_ref, d1_ref, b_ref, o_ref):
    deg = d0_ref[...] + d1_ref[...] + 1.0
    dis = lax.rsqrt(deg)
    o_ref[...] = dis * (q_ref[0, ...] + g_ref[...]) + b_ref[...]


def _final(q, g, d0, d1, b, blk):
    npd, feat = g.shape
    hblk = (npd // NC) // blk

    return pl.pallas_call(
        _final_body,
        grid=(npd // blk,),
        in_specs=[
            pl.BlockSpec((1, blk, feat),
                         lambda i: (i // hblk, i % hblk, 0)),
            pl.BlockSpec((blk, feat), lambda i: (i, 0)),
            pl.BlockSpec((blk, 1), lambda i: (i, 0)),
            pl.BlockSpec((blk, 1), lambda i: (i, 0)),
            pl.BlockSpec((feat,), lambda i: (0,)),
        ],
        out_specs=pl.BlockSpec((blk, feat), lambda i: (i, 0)),
        out_shape=jax.ShapeDtypeStruct((npd, feat), jnp.float32),
    )(q, g, d0, d1, b)


def kernel(x, edge_index, W1, b1, W2, b2):
    n, d = x.shape
    e = edge_index.shape[1]
    npad = _pad_to(n, 1024)
    ept = _pad_to(e, NW * CHUNK * 2) // NW
    epad = ept * NW
    cpt = ept // CHUNK

    ei = edge_index
    if e % CHUNK:
        extra = CHUNK - e % CHUNK
        ei = jnp.concatenate(
            [ei, jnp.stack([(jnp.arange(extra, dtype=jnp.int32) * 53) % n,
                            jnp.full((extra,), n, jnp.int32)])], axis=1)
        e = e + extra

    nblk = cpt // NB
    sa, sl, dp, dl = _make_prep(e, epad, n, npad)(ei)
    src3 = sa.reshape(NW * nblk, NB, CHUNK)
    srcloc = sl.reshape(NC * NW * nblk, NB, CHUNK)
    dst3 = dp.reshape(NW * nblk, NB, CHUNK)
    dst2 = dp.reshape(NW * (ept // (NB * CHUNK)), NB * CHUNK)
    dstloc = dl.reshape(NC * NW * nblk, NB, CHUNK)

    deg_parts = _make_deg_kernel(npad, ept)(dst2)
    d0 = deg_parts[0].reshape(npad, 1)
    d1 = deg_parts[1].reshape(npad, 1)

    blk = 1024
    g1d = _mm_scale(x, W1, d0, d1, npad)
    g1s = g1d.reshape(2 * npad, W1.shape[1])

    agg1 = _make_agg_kernel(npad, W1.shape[1], cpt)(g1s, src3, dst3)
    g2d = _mid(agg1[0], agg1[1], g1d[0], d0, d1, b1, W2, blk)
    g2s = g2d.reshape(2 * npad, W2.shape[1])

    agg2 = _make_agg_split_kernel(npad, W2.shape[1], cpt)(g2s, srcloc, dstloc)
    out = _final(agg2, g2d[0], d0, d1, b2, blk)
    return out[:n]
